# baseline (speedup 1.0000x reference)
import functools

import jax
import jax.numpy as jnp
from jax import lax
from jax.experimental import pallas as pl
from jax.experimental.pallas import tpu as pltpu
from jax.experimental.pallas import tpu_sc as plsc

_NUM_ENTITIES = 100000
_NUM_RELATIONS = 512
_EMB_DIM = 64
_NC = 2
_NS = 16
_NW = _NC * _NS
_CHUNK = 312
_NFULL = _NUM_ENTITIES // _CHUNK
_ROUNDS = _NFULL // _NW
_TAIL = _NUM_ENTITIES - _NFULL * _CHUNK
_REL_PER_W = _NUM_RELATIONS // _NW


def _sc_copy_body(ent_in, rel_in, ent_out, rel_out,
                  buf0, buf1, tbuf, rbuf, sin0, sin1, sout0, sout1):
    wid = lax.axis_index("s") * _NC + lax.axis_index("c")
    bufs = (buf0, buf1)
    in_sems = (sin0, sin1)
    out_sems = (sout0, sout1)

    out_copies = [None] * _ROUNDS
    for k in range(_ROUNDS):
        b = k % 2
        if k >= 2:
            out_copies[k - 2].wait()
        rows = pl.ds((wid + k * _NW) * _CHUNK, _CHUNK)
        in_copy = pltpu.make_async_copy(ent_in.at[rows], bufs[b], in_sems[b])
        in_copy.start()
        in_copy.wait()
        out_copies[k] = pltpu.make_async_copy(bufs[b], ent_out.at[rows],
                                              out_sems[b])
        out_copies[k].start()

    rrows = pl.ds(wid * _REL_PER_W, _REL_PER_W)
    pltpu.sync_copy(rel_in.at[rrows], rbuf)
    pltpu.sync_copy(rbuf, rel_out.at[rrows])

    @pl.when(wid == 0)
    def _copy_tail():
        trows = pl.ds(_NFULL * _CHUNK, _TAIL)
        pltpu.sync_copy(ent_in.at[trows], tbuf)
        pltpu.sync_copy(tbuf, ent_out.at[trows])

    out_copies[_ROUNDS - 2].wait()
    out_copies[_ROUNDS - 1].wait()


@jax.jit
def _sc_copy(entity_emb, rel_emb):
    mesh = plsc.VectorSubcoreMesh(core_axis_name="c", subcore_axis_name="s")
    k = pl.kernel(
        _sc_copy_body,
        out_type=[
            jax.ShapeDtypeStruct((_NUM_ENTITIES, _EMB_DIM), jnp.float32),
            jax.ShapeDtypeStruct((_NUM_RELATIONS, _EMB_DIM), jnp.float32),
        ],
        mesh=mesh,
        scratch_types=[
            pltpu.VMEM((_CHUNK, _EMB_DIM), jnp.float32),
            pltpu.VMEM((_CHUNK, _EMB_DIM), jnp.float32),
            pltpu.VMEM((_TAIL, _EMB_DIM), jnp.float32),
            pltpu.VMEM((_REL_PER_W, _EMB_DIM), jnp.float32),
            pltpu.SemaphoreType.DMA,
            pltpu.SemaphoreType.DMA,
            pltpu.SemaphoreType.DMA,
            pltpu.SemaphoreType.DMA,
        ],
    )
    return k(entity_emb, rel_emb)


def kernel(x_dict, edge_index, entity_emb, rel_emb):
    entity_out, rel_out = _sc_copy(entity_emb, rel_emb)
    return (entity_out, rel_out)


# speedup vs baseline: 1.2151x; 1.2151x over previous
import jax
import jax.numpy as jnp
from jax.experimental import pallas as pl
from jax.experimental.pallas import tpu as pltpu

_CHUNK = 1000
_K = 10
_R = 10


def _copy_body(ent_in, rel_in, ent_out, rel_out, bufs, rbuf,
               in_sems, out_sems, rsem):
    rin = pltpu.make_async_copy(rel_in, rbuf, rsem)
    rin.start()

    def rows(c, r):
        return pl.ds((r * _K + c) * _CHUNK, _CHUNK)

    in_copies, out_copies = {}, {}
    for c in range(_K):
        in_copies[(c, 0)] = pltpu.make_async_copy(
            ent_in.at[rows(c, 0)], bufs.at[c, 0], in_sems.at[c, 0])
        in_copies[(c, 0)].start()
    for r in range(_R):
        b, nb = r % 2, (r + 1) % 2
        for c in range(_K):
            if r + 1 < _R:
                if r >= 1:
                    out_copies[(c, r - 1)].wait()
                in_copies[(c, r + 1)] = pltpu.make_async_copy(
                    ent_in.at[rows(c, r + 1)], bufs.at[c, nb],
                    in_sems.at[c, nb])
                in_copies[(c, r + 1)].start()
            in_copies[(c, r)].wait()
            out_copies[(c, r)] = pltpu.make_async_copy(
                bufs.at[c, b], ent_out.at[rows(c, r)], out_sems.at[c, b])
            out_copies[(c, r)].start()

    rin.wait()
    rout = pltpu.make_async_copy(rbuf, rel_out, rsem)
    rout.start()
    for c in range(_K):
        out_copies[(c, _R - 2)].wait()
        out_copies[(c, _R - 1)].wait()
    rout.wait()


def kernel(x_dict, edge_index, entity_emb, rel_emb):
    entity_out, rel_out = pl.pallas_call(
        _copy_body,
        in_specs=[
            pl.BlockSpec(memory_space=pl.ANY),
            pl.BlockSpec(memory_space=pl.ANY),
        ],
        out_specs=[
            pl.BlockSpec(memory_space=pl.ANY),
            pl.BlockSpec(memory_space=pl.ANY),
        ],
        scratch_shapes=[
            pltpu.VMEM((_K, 2, _CHUNK, 64), jnp.float32),
            pltpu.VMEM((512, 64), jnp.float32),
            pltpu.SemaphoreType.DMA((_K, 2)),
            pltpu.SemaphoreType.DMA((_K, 2)),
            pltpu.SemaphoreType.DMA,
        ],
        out_shape=[
            jax.ShapeDtypeStruct(entity_emb.shape, entity_emb.dtype),
            jax.ShapeDtypeStruct(rel_emb.shape, rel_emb.dtype),
        ],
    )(entity_emb, rel_emb)
    return (entity_out, rel_out)
